# Initial kernel scaffold; baseline (speedup 1.0000x reference)
import functools

import jax
import jax.numpy as jnp
from jax import lax
from jax.experimental import pallas as pl
from jax.experimental.pallas import tpu as pltpu
from jax.experimental.pallas import tpu_sc as plsc

DIM = 64
BATCH = 16384
NEG = 20
K1 = NEG + 1
LANES = 16
NC = 2
NS = 16
NW = NC * NS
B_PER_W = BATCH // NW
CB = 16
NCHUNK = B_PER_W // CB
KROWS = CB * K1
KSPLIT = 3
KG = KROWS // KSPLIT
CBK = K1 * LANES
NGROUPS = BATCH // CB


def _sc_body(center_hbm, combo_hbm, in_emb_hbm, out_emb_hbm, scores_hbm,
             cidx_v, kidx_v, crow_v, krow_v, scores_v, sem):
    wid = lax.axis_index("s") * NC + lax.axis_index("c")
    lane_iota = lax.iota(jnp.int32, LANES)

    def chunk_body(c, carry):
        base = wid * B_PER_W + c * CB
        pltpu.sync_copy(center_hbm.at[pl.ds(base, CB)], cidx_v)
        pltpu.sync_copy(combo_hbm.at[pl.ds(base * K1, KROWS)], kidx_v)
        handles = [pltpu.async_copy(in_emb_hbm.at[cidx_v], crow_v, sem)]
        for j in range(KSPLIT):
            handles.append(pltpu.async_copy(
                out_emb_hbm.at[kidx_v.at[pl.ds(j * KG, KG)]],
                krow_v.at[pl.ds(j * KG, KG)], sem))
        for h in handles:
            h.wait()

        def item_body(i, vecs):
            cs = [crow_v[i, pl.ds(q * LANES, LANES)] for q in range(DIM // LANES)]
            out = []
            for k in range(K1):
                r = i * K1 + k
                acc = cs[0] * krow_v[r, pl.ds(0, LANES)]
                for q in range(1, DIM // LANES):
                    acc = acc + cs[q] * krow_v[r, pl.ds(q * LANES, LANES)]
                s = jnp.sum(acc)
                out.append(jnp.where(lane_iota == i, s, vecs[k]))
            return tuple(out)

        vecs = lax.fori_loop(
            0, CB, item_body,
            tuple(jnp.zeros((LANES,), jnp.float32) for _ in range(K1)))
        for k in range(K1):
            scores_v[pl.ds(k * LANES, LANES)] = vecs[k]
        pltpu.sync_copy(scores_v, scores_hbm.at[wid * NCHUNK + c])
        return carry

    lax.fori_loop(0, NCHUNK, chunk_body, 0)


_sc_scores = functools.partial(
    pl.kernel,
    out_type=jax.ShapeDtypeStruct((NGROUPS, CBK), jnp.float32),
    mesh=plsc.VectorSubcoreMesh(core_axis_name="c", subcore_axis_name="s"),
    scratch_types=[
        pltpu.VMEM((CB,), jnp.int32),
        pltpu.VMEM((KROWS,), jnp.int32),
        pltpu.VMEM((CB, DIM), jnp.float32),
        pltpu.VMEM((KROWS, DIM), jnp.float32),
        pltpu.VMEM((CBK,), jnp.float32),
        pltpu.SemaphoreType.DMA,
    ],
)(_sc_body)


def _tc_loss_body(scores_ref, out_ref):
    x = scores_ref[...]
    r = lax.broadcasted_iota(jnp.int32, x.shape, 0)
    c = lax.broadcasted_iota(jnp.int32, x.shape, 1)
    k = (r * (x.shape[1] // LANES) + c // LANES) % K1
    z = jnp.where(k == 0, x, -x)
    loss = -jnp.log(jax.nn.sigmoid(z) + 1e-10)
    out_ref[0, 0] = jnp.sum(loss) * (1.0 / BATCH)


def kernel(center_words, context_words, negative_samples, in_emb, out_emb):
    center = center_words.astype(jnp.int32)
    combo = jnp.concatenate(
        [context_words[:, None], negative_samples], axis=1
    ).reshape(-1).astype(jnp.int32)
    scores = _sc_scores(center, combo, in_emb, out_emb)
    flat = scores.reshape(NGROUPS * CBK // 128, 128)
    loss = pl.pallas_call(
        _tc_loss_body,
        out_shape=jax.ShapeDtypeStruct((1, 1), jnp.float32),
        out_specs=pl.BlockSpec(memory_space=pltpu.SMEM),
    )(flat)
    return loss[0, 0]


# speedup vs baseline: 5.1986x; 5.1986x over previous
import functools

import jax
import jax.numpy as jnp
from jax import lax
from jax.experimental import pallas as pl
from jax.experimental.pallas import tpu as pltpu
from jax.experimental.pallas import tpu_sc as plsc

DIM = 64
BATCH = 16384
NEG = 20
K1 = NEG + 1
LANES = 16
NC = 2
NS = 16
NW = NC * NS
B_PER_W = BATCH // NW
CB = 16
NCHUNK = B_PER_W // CB
KROWS = CB * K1
KSPLIT = 3
KG = KROWS // KSPLIT
CBK = K1 * LANES
NGROUPS = BATCH // CB


def _sc_body(center_hbm, combo_hbm, in_emb_hbm, out_emb_hbm, scores_hbm,
             cidx_v, kidx_v, crow_v, krow_v, scores_v, sem):
    wid = lax.axis_index("s") * NC + lax.axis_index("c")
    lane_iota = lax.iota(jnp.int32, LANES)

    def chunk_body(c, carry):
        base = wid * B_PER_W + c * CB
        pltpu.sync_copy(center_hbm.at[pl.ds(base, CB)], cidx_v)
        pltpu.sync_copy(combo_hbm.at[pl.ds(base * K1, KROWS)], kidx_v)
        handles = [pltpu.async_copy(in_emb_hbm.at[cidx_v], crow_v, sem)]
        for j in range(KSPLIT):
            handles.append(pltpu.async_copy(
                out_emb_hbm.at[kidx_v.at[pl.ds(j * KG, KG)]],
                krow_v.at[pl.ds(j * KG, KG)], sem))
        for h in handles:
            h.wait()

        def item_body(i, vecs):
            cs = [crow_v[i, pl.ds(q * LANES, LANES)] for q in range(DIM // LANES)]
            out = []
            for k in range(K1):
                r = i * K1 + k
                acc = cs[0] * krow_v[r, pl.ds(0, LANES)]
                for q in range(1, DIM // LANES):
                    acc = acc + cs[q] * krow_v[r, pl.ds(q * LANES, LANES)]
                s = jnp.sum(acc)
                out.append(jnp.where(lane_iota == i, s, vecs[k]))
            return tuple(out)

        vecs = lax.fori_loop(
            0, CB, item_body,
            tuple(jnp.zeros((LANES,), jnp.float32) for _ in range(K1)))
        for k in range(K1):
            scores_v[pl.ds(k * LANES, LANES)] = vecs[k]
        pltpu.sync_copy(scores_v, scores_hbm.at[wid * NCHUNK + c])
        return carry

    lax.fori_loop(0, NCHUNK, chunk_body, 0)


_sc_scores = functools.partial(
    pl.kernel,
    out_type=jax.ShapeDtypeStruct((NGROUPS, CBK), jnp.float32),
    mesh=plsc.VectorSubcoreMesh(core_axis_name="c", subcore_axis_name="s"),
    compiler_params=pltpu.CompilerParams(
        needs_layout_passes=False, use_tc_tiling_on_sc=False),
    scratch_types=[
        pltpu.VMEM((CB,), jnp.int32),
        pltpu.VMEM((KROWS,), jnp.int32),
        pltpu.VMEM((CB, DIM), jnp.float32),
        pltpu.VMEM((KROWS, DIM), jnp.float32),
        pltpu.VMEM((CBK,), jnp.float32),
        pltpu.SemaphoreType.DMA,
    ],
)(_sc_body)


def _tc_loss_body(scores_ref, out_ref):
    x = scores_ref[...]
    r = lax.broadcasted_iota(jnp.int32, x.shape, 0)
    c = lax.broadcasted_iota(jnp.int32, x.shape, 1)
    k = (r * (x.shape[1] // LANES) + c // LANES) % K1
    z = jnp.where(k == 0, x, -x)
    loss = -jnp.log(jax.nn.sigmoid(z) + 1e-10)
    out_ref[0, 0] = jnp.sum(loss) * (1.0 / BATCH)


def kernel(center_words, context_words, negative_samples, in_emb, out_emb):
    center = center_words.astype(jnp.int32)
    combo = jnp.concatenate(
        [context_words[:, None], negative_samples], axis=1
    ).reshape(-1).astype(jnp.int32)
    scores = _sc_scores(center, combo, in_emb, out_emb)
    flat = scores.reshape(NGROUPS * CBK // 128, 128)
    loss = pl.pallas_call(
        _tc_loss_body,
        out_shape=jax.ShapeDtypeStruct((1, 1), jnp.float32),
        out_specs=pl.BlockSpec(memory_space=pltpu.SMEM),
    )(flat)
    return loss[0, 0]
